# baseline (speedup 1.0000x reference)
import functools

import jax
import jax.numpy as jnp
import numpy as np
from jax.experimental import pallas as pl
from jax.experimental.pallas import tpu as pltpu
from jax.sharding import Mesh, NamedSharding, PartitionSpec as P

try:
    from jax.experimental.shard_map import shard_map as _shard_map
except ImportError:
    _shard_map = jax.shard_map


def _expand_body(pos_ref, w_ref, out_ref):
    C = out_ref.shape[2]
    D = w_ref.shape[1]
    max_depth = C // (2 * D)

    c = jax.lax.broadcasted_iota(jnp.int32, (1, 1, C), 2)
    k = c // D
    t = k // 2
    par = k % 2
    e = k % max_depth

    w = jnp.tanh(w_ref[...])
    scale = jnp.sqrt((1.0 - w * w) * (D / 2.0))
    wt = jnp.concatenate([w] * (2 * max_depth), axis=1)
    st = jnp.concatenate([scale] * (2 * max_depth), axis=1)
    w2 = wt * wt
    w4 = w2 * w2
    w8 = w4 * w4
    e2 = e[0]
    v = st
    v = v * jnp.where((e2 & 1) != 0, wt, 1.0)
    v = v * jnp.where((e2 & 2) != 0, w2, 1.0)
    v = v * jnp.where((e2 & 4) != 0, w4, 1.0)
    v = v * jnp.where((e2 & 8) != 0, w8, 1.0)
    v = v[None]

    m = jnp.maximum(pos_ref[...], 1)
    sh = jnp.right_shift(m[:, :, None], t)
    bit = (sh >= 2) & ((sh & 1) == par)
    out_ref[...] = jnp.where(bit, v, 0.0)


@functools.partial(jax.jit, static_argnames=("block_b", "word_len"))
def _expand(positions, weight_row, block_b=64, word_len=20):
    b, s = positions.shape
    d = weight_row.shape[1]
    c = word_len * d
    grid = (b // block_b,)
    return pl.pallas_call(
        _expand_body,
        grid=grid,
        in_specs=[
            pl.BlockSpec((block_b, s), lambda i: (i, 0)),
            pl.BlockSpec((1, d), lambda i: (0, 0)),
        ],
        out_specs=pl.BlockSpec((block_b, s, c), lambda i: (i, 0, 0)),
        out_shape=jax.ShapeDtypeStruct((b, s, c), jnp.float32),
        compiler_params=pltpu.CompilerParams(
            dimension_semantics=("arbitrary",),
        ),
    )(positions, weight_row)


def kernel(positions, weight, paths):
    d = weight.shape[0]
    word_len = paths.shape[1]
    b = positions.shape[0]
    weight_row = weight.reshape(1, d)
    devs = jax.devices()
    n_dev = 2 if (len(devs) >= 2 and b % 2 == 0) else 1
    if n_dev == 1:
        return _expand(positions, weight_row, word_len=word_len)
    mesh = Mesh(np.asarray(devs[:2]), ("x",))
    pos_sh = jax.device_put(positions, NamedSharding(mesh, P("x", None)))
    w_sh = jax.device_put(weight_row, NamedSharding(mesh, P(None, None)))
    f = _shard_map(
        functools.partial(_expand, word_len=word_len),
        mesh=mesh,
        in_specs=(P("x", None), P(None, None)),
        out_specs=P("x", None, None),
        check_rep=False,
    )
    return f(pos_sh, w_sh)


# speedup vs baseline: 3.4270x; 3.4270x over previous
import functools

import jax
import jax.numpy as jnp
import numpy as np
from jax.experimental import pallas as pl
from jax.experimental.pallas import tpu as pltpu
from jax.sharding import Mesh, NamedSharding, PartitionSpec as P

try:
    from jax.experimental.shard_map import shard_map as _shard_map
except ImportError:
    _shard_map = jax.shard_map


def _expand_body(pos_ref, w_ref, out_ref):
    C = out_ref.shape[2]
    D = w_ref.shape[1]
    max_depth = C // (2 * D)

    c = jax.lax.broadcasted_iota(jnp.int32, (1, 1, C), 2)
    k = c // D
    t = k // 2
    par = k % 2
    e = k % max_depth

    w = jnp.tanh(w_ref[...])
    scale = jnp.sqrt((1.0 - w * w) * (D / 2.0))
    wt = jnp.concatenate([w] * (2 * max_depth), axis=1)
    st = jnp.concatenate([scale] * (2 * max_depth), axis=1)
    w2 = wt * wt
    w4 = w2 * w2
    w8 = w4 * w4
    e2 = e[0]
    v = st
    v = v * jnp.where((e2 & 1) != 0, wt, 1.0)
    v = v * jnp.where((e2 & 2) != 0, w2, 1.0)
    v = v * jnp.where((e2 & 4) != 0, w4, 1.0)
    v = v * jnp.where((e2 & 8) != 0, w8, 1.0)
    v = v[None]

    m = jnp.maximum(pos_ref[...], 1)
    sh = jnp.right_shift(m[:, :, None], t)
    bit = (sh >= 2) & ((sh & 1) == par)
    out_ref[...] = jnp.where(bit, v, 0.0)


@functools.partial(jax.jit, static_argnames=("block_b", "word_len"))
def _expand(positions, weight_row, block_b=64, word_len=20):
    b, s = positions.shape
    d = weight_row.shape[1]
    c = word_len * d
    grid = (b // block_b,)
    return pl.pallas_call(
        _expand_body,
        grid=grid,
        in_specs=[
            pl.BlockSpec((block_b, s), lambda i: (i, 0)),
            pl.BlockSpec((1, d), lambda i: (0, 0)),
        ],
        out_specs=pl.BlockSpec((block_b, s, c), lambda i: (i, 0, 0)),
        out_shape=jax.ShapeDtypeStruct((b, s, c), jnp.float32),
        compiler_params=pltpu.CompilerParams(
            dimension_semantics=("arbitrary",),
        ),
    )(positions, weight_row)


def kernel(positions, weight, paths):
    d = weight.shape[0]
    word_len = paths.shape[1]
    b = positions.shape[0]
    weight_row = weight.reshape(1, d)
    devs = jax.devices()
    n_dev = 1
    if n_dev == 1:
        return _expand(positions, weight_row, word_len=word_len)
    mesh = Mesh(np.asarray(devs[:2]), ("x",))
    pos_sh = jax.device_put(positions, NamedSharding(mesh, P("x", None)))
    w_sh = jax.device_put(weight_row, NamedSharding(mesh, P(None, None)))
    f = _shard_map(
        functools.partial(_expand, word_len=word_len),
        mesh=mesh,
        in_specs=(P("x", None), P(None, None)),
        out_specs=P("x", None, None),
        check_rep=False,
    )
    return f(pos_sh, w_sh)


# speedup vs baseline: 10.7266x; 3.1300x over previous
import functools

import jax
import jax.numpy as jnp
from jax.experimental import pallas as pl
from jax.experimental.pallas import tpu as pltpu


def _expand_body(pos_ref, w_ref, out_ref):
    C = out_ref.shape[2]
    D = w_ref.shape[1]
    max_depth = C // (2 * D)

    c = jax.lax.broadcasted_iota(jnp.int32, (1, 1, C), 2)
    k = c // D
    t = k // 2
    par = k % 2
    e = k % max_depth

    w = jnp.tanh(w_ref[...])
    scale = jnp.sqrt((1.0 - w * w) * (D / 2.0))
    wt = jnp.concatenate([w] * (2 * max_depth), axis=1)
    st = jnp.concatenate([scale] * (2 * max_depth), axis=1)
    w2 = wt * wt
    w4 = w2 * w2
    w8 = w4 * w4
    e2 = e[0]
    v = st
    v = v * jnp.where((e2 & 1) != 0, wt, 1.0)
    v = v * jnp.where((e2 & 2) != 0, w2, 1.0)
    v = v * jnp.where((e2 & 4) != 0, w4, 1.0)
    v = v * jnp.where((e2 & 8) != 0, w8, 1.0)
    v = v[None]

    m = jnp.maximum(pos_ref[...], 1)
    sh = jnp.right_shift(m[:, :, None], t)
    bit = (sh >= 2) & ((sh & 1) == par)
    out_ref[...] = jnp.where(bit, v, 0.0)


@functools.partial(jax.jit, static_argnames=("block_b", "word_len"))
def _expand(pos_t, weight_row, block_b=128, word_len=20):
    s, b = pos_t.shape
    d = weight_row.shape[1]
    c = word_len * d
    grid = (b // block_b,)
    return pl.pallas_call(
        _expand_body,
        grid=grid,
        in_specs=[
            pl.BlockSpec((s, block_b), lambda i: (0, i)),
            pl.BlockSpec((1, d), lambda i: (0, 0)),
        ],
        out_specs=pl.BlockSpec((s, block_b, c), lambda i: (0, i, 0)),
        out_shape=jax.ShapeDtypeStruct((s, b, c), jnp.float32),
        compiler_params=pltpu.CompilerParams(
            dimension_semantics=("arbitrary",),
        ),
    )(pos_t, weight_row)


def kernel(positions, weight, paths):
    d = weight.shape[0]
    word_len = paths.shape[1]
    weight_row = weight.reshape(1, d)
    out_t = _expand(positions.T, weight_row, word_len=word_len)
    return jnp.transpose(out_t, (1, 0, 2))
